# baseline (speedup 1.0000x reference)
import functools

import jax
import jax.numpy as jnp
from jax import lax
from jax.experimental import pallas as pl
from jax.experimental.pallas import tpu as pltpu
from jax.experimental.pallas import tpu_sc as plsc

NUM_NEG = 5
ROW_CHUNK = 128


def _sc_gather(table, idx, num_rows):
    info = plsc.get_sparse_core_info()
    nc, ns = info.num_cores, info.num_subcores
    nw = nc * ns
    emb = table.shape[1]
    r_per_w = num_rows // nw
    n_ch = r_per_w // ROW_CHUNK
    mesh = plsc.VectorSubcoreMesh(core_axis_name="c", subcore_axis_name="s")

    @functools.partial(
        pl.kernel,
        mesh=mesh,
        out_type=jax.ShapeDtypeStruct((num_rows, emb), jnp.float32),
        compiler_params=pltpu.CompilerParams(use_tc_tiling_on_sc=False),
        scratch_types=[
            pltpu.VMEM((r_per_w,), jnp.int32),
            pltpu.VMEM((ROW_CHUNK, emb), jnp.float32),
            pltpu.SemaphoreType.DMA,
        ],
    )
    def k(table_hbm, idx_hbm, out_hbm, idx_v, rows_v, sem):
        wid = lax.axis_index("s") * nc + lax.axis_index("c")
        base = wid * r_per_w
        pltpu.sync_copy(idx_hbm.at[pl.ds(base, r_per_w)], idx_v)

        def body(ch, carry):
            off = ch * ROW_CHUNK
            pltpu.async_copy(
                table_hbm.at[idx_v.at[pl.ds(off, ROW_CHUNK)]], rows_v, sem
            ).wait()
            pltpu.sync_copy(rows_v, out_hbm.at[pl.ds(base + off, ROW_CHUNK)])
            return carry

        lax.fori_loop(0, n_ch, body, 0)

    return k(table, idx)


def _tc_loss_sum(rows3, hidden):
    b_total, emb = hidden.shape
    bb = 2048
    grid = b_total // bb

    def body(r_ref, h_ref, o_ref):
        r = r_ref[...]
        h = h_ref[...]
        s = jnp.sum(r * h[None, :, :], axis=2)
        sign = jnp.where(
            lax.broadcasted_iota(jnp.int32, (1 + NUM_NEG, 1), 0) == 0, 1.0, -1.0
        )
        tot = jnp.sum(jax.nn.log_sigmoid(s * sign))

        @pl.when(pl.program_id(0) == 0)
        def _():
            o_ref[...] = jnp.zeros_like(o_ref)

        o_ref[...] = o_ref[...] + tot

    out = pl.pallas_call(
        body,
        grid=(grid,),
        in_specs=[
            pl.BlockSpec((1 + NUM_NEG, bb, emb), lambda i: (0, i, 0)),
            pl.BlockSpec((bb, emb), lambda i: (i, 0)),
        ],
        out_specs=pl.BlockSpec((1, 1), lambda i: (0, 0)),
        out_shape=jax.ShapeDtypeStruct((1, 1), jnp.float32),
    )(rows3, hidden)
    return out[0, 0]


def kernel(hidden, positives, table):
    b, emb = hidden.shape
    vocab = table.shape[0]
    neg_key = jax.random.key(42)
    negatives = jax.random.randint(neg_key, (b, NUM_NEG), 1, vocab - 1, dtype=jnp.int32)
    idx = jnp.concatenate([positives, jnp.transpose(negatives).reshape(-1)])
    rows = _sc_gather(table, idx, (1 + NUM_NEG) * b)
    rows3 = rows.reshape(1 + NUM_NEG, b, emb)
    total = _tc_loss_sum(rows3, hidden)
    return -total / b
